# baseline (speedup 1.0000x reference)
import functools

import jax
import jax.numpy as jnp
from jax import lax
from jax.experimental import pallas as pl
from jax.experimental.pallas import tpu as pltpu
from jax.experimental.pallas import tpu_sc as plsc

N = 10000
NP = 10240
D = 128
E = 320000
NC = 2
NS = 16
NW = NC * NS
KB = 128
E_PAD = 327680
CH = 40
NCHUNK = E_PAD // (CH * KB)
CPT0 = 3
CPT1 = 1
RPT = NP // NS
PAD_SRC = N
PAD_DST = NP - 1

DW = 16
DNP = 4 * NP
KD = 128
NBD = 320
E_DPAD = NW * NBD * KD
DRPT = DNP // NS

_mesh = plsc.VectorSubcoreMesh(core_axis_name="c", subcore_axis_name="s")


def _fill(ref, rows, value):
    cols = ref.shape[1] // 16
    vec = jnp.full((16,), value, jnp.float32)

    def body(i, _):
        for k in range(cols):
            ref[i, pl.ds(k * 16, 16)] = vec
        return 0

    lax.fori_loop(0, rows, body, 0)


@functools.partial(
    pl.kernel,
    out_type=jax.ShapeDtypeStruct((NC, NS, DNP), jnp.float32),
    mesh=_mesh,
    scratch_types=[
        pltpu.VMEM((DNP,), jnp.float32),
        pltpu.VMEM((NBD, KD), jnp.int32),
    ],
    compiler_params=pltpu.CompilerParams(needs_layout_passes=False),
)
def _deg_kernel(idx_hbm, out, counts, idxv):
    c = lax.axis_index("c")
    s = lax.axis_index("s")
    wid = c * NS + s

    zero16 = jnp.zeros((16,), jnp.float32)

    def z(i, _):
        counts[pl.ds(i * 16, 16)] = zero16
        return 0

    lax.fori_loop(0, DNP // 16, z, 0)

    pltpu.sync_copy(idx_hbm.at[wid], idxv)
    ones16 = jnp.ones((16,), jnp.float32)

    def batch(j, _):
        for k in range(KD // 16):
            idx = idxv[j, pl.ds(k * 16, 16)]
            plsc.addupdate_scatter(counts, [idx], ones16)
        return 0

    lax.fori_loop(0, NBD, batch, 0)
    pltpu.sync_copy(counts, out.at[c, s])


@functools.partial(
    pl.kernel,
    out_type=jax.ShapeDtypeStruct((2, NC, NS, RPT, D), jnp.float32),
    mesh=_mesh,
    scratch_types=[
        pltpu.VMEM_SHARED((NP, D), jnp.float32),
        pltpu.VMEM((CH, KB), jnp.int32),
        pltpu.VMEM((CH, KB), jnp.int32),
        pltpu.VMEM((KB, D), jnp.float32),
        pltpu.VMEM((KB, D), jnp.float32),
        pltpu.SemaphoreType.DMA,
        pltpu.SemaphoreType.DMA,
        pltpu.SemaphoreType.DMA,
        pltpu.SemaphoreType.DMA,
    ],
)
def _spmm_kernel(xns0, xns1, src0, dst0, src1, dst1, out,
                 acc, srcv, dstv, rb0, rb1, gs0, gs1, ss0, ss1):
    c = lax.axis_index("c")
    s = lax.axis_index("s")
    cbase = jnp.where(c == 0, CPT0 * s, CPT0 * NS + CPT1 * s)
    ccnt = jnp.where(c == 0, CPT0, CPT1)

    for rel, (xns, srch, dsth) in enumerate(
            ((xns0, src0, dst0), (xns1, src1, dst1))):
        with jax.named_scope(f"zero{rel}"):
            _fill(rb0, KB, 0.0)
            for r in range(RPT // KB):
                pltpu.sync_copy(rb0, acc.at[pl.ds(s * RPT + r * KB, KB)])
            plsc.subcore_barrier()

        def chunk(i, _):
            pltpu.sync_copy(srch.at[cbase + i], srcv)
            pltpu.sync_copy(dsth.at[cbase + i], dstv)

            pltpu.async_copy(xns.at[srcv.at[0]], rb0, gs0)
            pltpu.async_copy(xns.at[srcv.at[1]], rb1, gs1)

            def pair(p, _):
                j0 = 2 * p
                j1 = 2 * p + 1
                pltpu.make_async_copy(xns.at[srcv.at[j0]], rb0, gs0).wait()
                sc0 = pltpu.async_copy(rb0, acc.at[dstv.at[j0]], ss0, add=True)
                pltpu.make_async_copy(xns.at[srcv.at[j1]], rb1, gs1).wait()
                sc1 = pltpu.async_copy(rb1, acc.at[dstv.at[j1]], ss1, add=True)
                sc0.wait()

                @pl.when(j0 + 2 < CH)
                def _():
                    pltpu.async_copy(xns.at[srcv.at[j0 + 2]], rb0, gs0)

                sc1.wait()

                @pl.when(j1 + 2 < CH)
                def _():
                    pltpu.async_copy(xns.at[srcv.at[j1 + 2]], rb1, gs1)

                return 0

            lax.fori_loop(0, CH // 2, pair, 0)
            return 0

        with jax.named_scope(f"edges{rel}"):
            lax.fori_loop(0, ccnt, chunk, 0)
            plsc.subcore_barrier()

        with jax.named_scope(f"rdout{rel}"):
            for r in range(RPT // KB):
                pltpu.sync_copy(acc.at[pl.ds(s * RPT + r * KB, KB)], rb0)
                pltpu.sync_copy(rb0, out.at[rel, c, s, pl.ds(r * KB, KB)])
            if rel == 0:
                plsc.subcore_barrier()


_R = 640


def _scale_body(x_ref, ns0_ref, ns1_ref, o0_ref, o1_ref):
    x = x_ref[...]
    o0_ref[...] = x * ns0_ref[...]
    o1_ref[...] = x * ns1_ref[...]


_scale_call = pl.pallas_call(
    _scale_body,
    grid=(NP // _R,),
    in_specs=[
        pl.BlockSpec((_R, D), lambda i: (i, 0)),
        pl.BlockSpec((_R, 1), lambda i: (i, 0)),
        pl.BlockSpec((_R, 1), lambda i: (i, 0)),
    ],
    out_specs=[pl.BlockSpec((_R, D), lambda i: (i, 0))] * 2,
    out_shape=[jax.ShapeDtypeStruct((NP, D), jnp.float32)] * 2,
)


def _layer1_body(a_ref, nd0_ref, nd1_ref, ns0_ref, ns1_ref,
                 w0_ref, w1_ref, b0_ref, b1_ref, o0_ref, o1_ref):
    s0 = (a_ref[0, 0] + a_ref[0, 1]) * nd0_ref[...]
    s1 = (a_ref[1, 0] + a_ref[1, 1]) * nd1_ref[...]
    h = (jnp.dot(s0, w0_ref[...], preferred_element_type=jnp.float32)
         + b0_ref[...]
         + jnp.dot(s1, w1_ref[...], preferred_element_type=jnp.float32)
         + b1_ref[...]) * 0.5
    h = jnp.maximum(h, 0.0)
    o0_ref[...] = h * ns0_ref[...]
    o1_ref[...] = h * ns1_ref[...]


_layer1_call = pl.pallas_call(
    _layer1_body,
    grid=(NP // _R,),
    in_specs=[
        pl.BlockSpec((2, NC, _R, D), lambda i: (0, 0, i, 0)),
        pl.BlockSpec((_R, 1), lambda i: (i, 0)),
        pl.BlockSpec((_R, 1), lambda i: (i, 0)),
        pl.BlockSpec((_R, 1), lambda i: (i, 0)),
        pl.BlockSpec((_R, 1), lambda i: (i, 0)),
        pl.BlockSpec((D, D), lambda i: (0, 0)),
        pl.BlockSpec((D, D), lambda i: (0, 0)),
        pl.BlockSpec((1, D), lambda i: (0, 0)),
        pl.BlockSpec((1, D), lambda i: (0, 0)),
    ],
    out_specs=[pl.BlockSpec((_R, D), lambda i: (i, 0))] * 2,
    out_shape=[jax.ShapeDtypeStruct((NP, D), jnp.float32)] * 2,
)


def _layer2_body(a_ref, nd0_ref, nd1_ref,
                 w0_ref, w1_ref, b0_ref, b1_ref, o_ref):
    s0 = (a_ref[0, 0] + a_ref[0, 1]) * nd0_ref[...]
    s1 = (a_ref[1, 0] + a_ref[1, 1]) * nd1_ref[...]
    o_ref[...] = (jnp.dot(s0, w0_ref[...], preferred_element_type=jnp.float32)
                  + b0_ref[...]
                  + jnp.dot(s1, w1_ref[...], preferred_element_type=jnp.float32)
                  + b1_ref[...]) * 0.5


_layer2_call = pl.pallas_call(
    _layer2_body,
    grid=(NP // _R,),
    in_specs=[
        pl.BlockSpec((2, NC, _R, D), lambda i: (0, 0, i, 0)),
        pl.BlockSpec((_R, 1), lambda i: (i, 0)),
        pl.BlockSpec((_R, 1), lambda i: (i, 0)),
        pl.BlockSpec((D, D), lambda i: (0, 0)),
        pl.BlockSpec((D, D), lambda i: (0, 0)),
        pl.BlockSpec((1, D), lambda i: (0, 0)),
        pl.BlockSpec((1, D), lambda i: (0, 0)),
    ],
    out_specs=pl.BlockSpec((_R, D), lambda i: (i, 0)),
    out_shape=jax.ShapeDtypeStruct((NP, D), jnp.float32),
)


def kernel(x, edge_index_r0, edge_index_r1, W1_r0, b1_r0, W1_r1, b1_r1,
           W2_r0, b2_r0, W2_r1, b2_r1):
    src0 = edge_index_r0[0].astype(jnp.int32)
    dst0 = edge_index_r0[1].astype(jnp.int32)
    src1 = edge_index_r1[0].astype(jnp.int32)
    dst1 = edge_index_r1[1].astype(jnp.int32)

    dpad = jnp.full((E_DPAD - 4 * E,), DNP - 1, jnp.int32)
    deg_idx = jnp.concatenate(
        [src0, dst0 + NP, src1 + 2 * NP, dst1 + 3 * NP, dpad]
    ).reshape(NW, NBD, KD)
    degp = _deg_kernel(deg_idx)
    deg = degp.sum((0, 1))
    rs = lax.rsqrt(jnp.maximum(deg, 1.0)).reshape(4, NP)
    ns0 = rs[0, :, None]
    nd0 = rs[1, :, None]
    ns1 = rs[2, :, None]
    nd1 = rs[3, :, None]

    epad_s = jnp.full((E_PAD - E,), PAD_SRC, jnp.int32)
    epad_d = jnp.full((E_PAD - E,), PAD_DST, jnp.int32)
    src0r = jnp.concatenate([src0, epad_s]).reshape(NCHUNK, CH, KB)
    dst0r = jnp.concatenate([dst0, epad_d]).reshape(NCHUNK, CH, KB)
    src1r = jnp.concatenate([src1, epad_s]).reshape(NCHUNK, CH, KB)
    dst1r = jnp.concatenate([dst1, epad_d]).reshape(NCHUNK, CH, KB)

    b1_r0_2d = b1_r0[None, :]
    b1_r1_2d = b1_r1[None, :]
    b2_r0_2d = b2_r0[None, :]
    b2_r1_2d = b2_r1[None, :]

    xp = jnp.pad(x, ((0, NP - N), (0, 0)))
    xns0, xns1 = _scale_call(xp, ns0, ns1)
    agg1 = _spmm_kernel(xns0, xns1, src0r, dst0r, src1r, dst1r)
    agg1 = agg1.reshape(2, NC, NP, D)
    hns0, hns1 = _layer1_call(agg1, nd0, nd1, ns0, ns1,
                              W1_r0, W1_r1, b1_r0_2d, b1_r1_2d)
    agg2 = _spmm_kernel(hns0, hns1, src0r, dst0r, src1r, dst1r)
    agg2 = agg2.reshape(2, NC, NP, D)
    outp = _layer2_call(agg2, nd0, nd1, W2_r0, W2_r1, b2_r0_2d, b2_r1_2d)
    return outp[:N]


# speedup vs baseline: 2.4708x; 2.4708x over previous
import functools

import jax
import jax.numpy as jnp
from jax import lax
from jax.experimental import pallas as pl
from jax.experimental.pallas import tpu as pltpu
from jax.experimental.pallas import tpu_sc as plsc

N = 10000
NP = 10240
D = 128
E = 320000
NC = 2
NS = 16
NW = NC * NS
KB = 128
E_PAD = 327680
CH = 40
NCHUNK = E_PAD // (CH * KB)
CPT0 = 2
CPT1 = 2
RPT = NP // NS
PAD_SRC = N
PAD_DST = NP - 1

DW = 16
DNP = 4 * NP
KD = 128
NBD = 320
E_DPAD = NW * NBD * KD
DRPT = DNP // NS

_mesh = plsc.VectorSubcoreMesh(core_axis_name="c", subcore_axis_name="s")


def _fill(ref, rows, value):
    cols = ref.shape[1] // 16
    vec = jnp.full((16,), value, jnp.float32)

    def body(i, _):
        for k in range(cols):
            ref[i, pl.ds(k * 16, 16)] = vec
        return 0

    lax.fori_loop(0, rows, body, 0)


@functools.partial(
    pl.kernel,
    out_type=jax.ShapeDtypeStruct((NC, NS, DNP), jnp.float32),
    mesh=_mesh,
    scratch_types=[
        pltpu.VMEM((DNP,), jnp.float32),
        pltpu.VMEM((NBD, KD), jnp.int32),
    ],
    compiler_params=pltpu.CompilerParams(needs_layout_passes=False),
)
def _deg_kernel(idx_hbm, out, counts, idxv):
    c = lax.axis_index("c")
    s = lax.axis_index("s")
    wid = c * NS + s

    zero16 = jnp.zeros((16,), jnp.float32)

    def z(i, _):
        counts[pl.ds(i * 16, 16)] = zero16
        return 0

    lax.fori_loop(0, DNP // 16, z, 0)

    pltpu.sync_copy(idx_hbm.at[wid], idxv)
    ones16 = jnp.ones((16,), jnp.float32)

    def batch(j, _):
        for k in range(KD // 16):
            idx = idxv[j, pl.ds(k * 16, 16)]
            plsc.addupdate_scatter(counts, [idx], ones16)
        return 0

    lax.fori_loop(0, NBD, batch, 0)
    pltpu.sync_copy(counts, out.at[c, s])


@functools.partial(
    pl.kernel,
    out_type=jax.ShapeDtypeStruct((2, NC, NS, RPT, D), jnp.float32),
    mesh=_mesh,
    scratch_types=[
        pltpu.VMEM_SHARED((NP, D), jnp.float32),
        pltpu.VMEM((CH, KB), jnp.int32),
        pltpu.VMEM((CH, KB), jnp.int32),
        pltpu.VMEM((KB, D), jnp.float32),
        pltpu.VMEM((KB, D), jnp.float32),
        pltpu.SemaphoreType.DMA,
        pltpu.SemaphoreType.DMA,
        pltpu.SemaphoreType.DMA,
        pltpu.SemaphoreType.DMA,
    ],
)
def _spmm_kernel(xns0, xns1, src0, dst0, src1, dst1, out,
                 acc, srcv, dstv, rb0, rb1, gs0, gs1, ss0, ss1):
    c = lax.axis_index("c")
    s = lax.axis_index("s")
    cbase = jnp.where(c == 0, CPT0 * s, CPT0 * NS + CPT1 * s)
    ccnt = jnp.where(c == 0, CPT0, CPT1)

    for rel, (xns, srch, dsth) in enumerate(
            ((xns0, src0, dst0), (xns1, src1, dst1))):
        with jax.named_scope(f"zero{rel}"):
            _fill(rb0, KB, 0.0)
            for r in range(RPT // KB):
                pltpu.sync_copy(rb0, acc.at[pl.ds(s * RPT + r * KB, KB)])
            plsc.subcore_barrier()

        def chunk(i, _):
            pltpu.sync_copy(srch.at[cbase + i], srcv)
            pltpu.sync_copy(dsth.at[cbase + i], dstv)

            pltpu.async_copy(xns.at[srcv.at[0]], rb0, gs0)
            pltpu.async_copy(xns.at[srcv.at[1]], rb1, gs1)

            def pair(p, _):
                j0 = 2 * p
                j1 = 2 * p + 1
                pltpu.make_async_copy(xns.at[srcv.at[j0]], rb0, gs0).wait()
                sc0 = pltpu.async_copy(rb0, acc.at[dstv.at[j0]], ss0, add=True)
                pltpu.make_async_copy(xns.at[srcv.at[j1]], rb1, gs1).wait()
                sc1 = pltpu.async_copy(rb1, acc.at[dstv.at[j1]], ss1, add=True)
                sc0.wait()

                @pl.when(j0 + 2 < CH)
                def _():
                    pltpu.async_copy(xns.at[srcv.at[j0 + 2]], rb0, gs0)

                sc1.wait()

                @pl.when(j1 + 2 < CH)
                def _():
                    pltpu.async_copy(xns.at[srcv.at[j1 + 2]], rb1, gs1)

                return 0

            lax.fori_loop(0, CH // 2, pair, 0)
            return 0

        with jax.named_scope(f"edges{rel}"):
            lax.fori_loop(0, ccnt, chunk, 0)
            plsc.subcore_barrier()

        with jax.named_scope(f"rdout{rel}"):
            for r in range(RPT // KB):
                pltpu.sync_copy(acc.at[pl.ds(s * RPT + r * KB, KB)], rb0)
                pltpu.sync_copy(rb0, out.at[rel, c, s, pl.ds(r * KB, KB)])
            if rel == 0:
                plsc.subcore_barrier()


_R = 640


def _scale_body(x_ref, ns0_ref, ns1_ref, o0_ref, o1_ref):
    x = x_ref[...]
    o0_ref[...] = x * ns0_ref[...]
    o1_ref[...] = x * ns1_ref[...]


_scale_call = pl.pallas_call(
    _scale_body,
    grid=(NP // _R,),
    in_specs=[
        pl.BlockSpec((_R, D), lambda i: (i, 0)),
        pl.BlockSpec((_R, 1), lambda i: (i, 0)),
        pl.BlockSpec((_R, 1), lambda i: (i, 0)),
    ],
    out_specs=[pl.BlockSpec((_R, D), lambda i: (i, 0))] * 2,
    out_shape=[jax.ShapeDtypeStruct((NP, D), jnp.float32)] * 2,
)


def _layer1_body(a_ref, nd0_ref, nd1_ref, ns0_ref, ns1_ref,
                 w0_ref, w1_ref, b0_ref, b1_ref, o0_ref, o1_ref):
    s0 = (a_ref[0, 0] + a_ref[0, 1]) * nd0_ref[...]
    s1 = (a_ref[1, 0] + a_ref[1, 1]) * nd1_ref[...]
    h = (jnp.dot(s0, w0_ref[...], preferred_element_type=jnp.float32)
         + b0_ref[...]
         + jnp.dot(s1, w1_ref[...], preferred_element_type=jnp.float32)
         + b1_ref[...]) * 0.5
    h = jnp.maximum(h, 0.0)
    o0_ref[...] = h * ns0_ref[...]
    o1_ref[...] = h * ns1_ref[...]


_layer1_call = pl.pallas_call(
    _layer1_body,
    grid=(NP // _R,),
    in_specs=[
        pl.BlockSpec((2, NC, _R, D), lambda i: (0, 0, i, 0)),
        pl.BlockSpec((_R, 1), lambda i: (i, 0)),
        pl.BlockSpec((_R, 1), lambda i: (i, 0)),
        pl.BlockSpec((_R, 1), lambda i: (i, 0)),
        pl.BlockSpec((_R, 1), lambda i: (i, 0)),
        pl.BlockSpec((D, D), lambda i: (0, 0)),
        pl.BlockSpec((D, D), lambda i: (0, 0)),
        pl.BlockSpec((1, D), lambda i: (0, 0)),
        pl.BlockSpec((1, D), lambda i: (0, 0)),
    ],
    out_specs=[pl.BlockSpec((_R, D), lambda i: (i, 0))] * 2,
    out_shape=[jax.ShapeDtypeStruct((NP, D), jnp.float32)] * 2,
)


def _layer2_body(a_ref, nd0_ref, nd1_ref,
                 w0_ref, w1_ref, b0_ref, b1_ref, o_ref):
    s0 = (a_ref[0, 0] + a_ref[0, 1]) * nd0_ref[...]
    s1 = (a_ref[1, 0] + a_ref[1, 1]) * nd1_ref[...]
    o_ref[...] = (jnp.dot(s0, w0_ref[...], preferred_element_type=jnp.float32)
                  + b0_ref[...]
                  + jnp.dot(s1, w1_ref[...], preferred_element_type=jnp.float32)
                  + b1_ref[...]) * 0.5


_layer2_call = pl.pallas_call(
    _layer2_body,
    grid=(NP // _R,),
    in_specs=[
        pl.BlockSpec((2, NC, _R, D), lambda i: (0, 0, i, 0)),
        pl.BlockSpec((_R, 1), lambda i: (i, 0)),
        pl.BlockSpec((_R, 1), lambda i: (i, 0)),
        pl.BlockSpec((D, D), lambda i: (0, 0)),
        pl.BlockSpec((D, D), lambda i: (0, 0)),
        pl.BlockSpec((1, D), lambda i: (0, 0)),
        pl.BlockSpec((1, D), lambda i: (0, 0)),
    ],
    out_specs=pl.BlockSpec((_R, D), lambda i: (i, 0)),
    out_shape=jax.ShapeDtypeStruct((NP, D), jnp.float32),
)


def kernel(x, edge_index_r0, edge_index_r1, W1_r0, b1_r0, W1_r1, b1_r1,
           W2_r0, b2_r0, W2_r1, b2_r1):
    src0 = edge_index_r0[0].astype(jnp.int32)
    dst0 = edge_index_r0[1].astype(jnp.int32)
    src1 = edge_index_r1[0].astype(jnp.int32)
    dst1 = edge_index_r1[1].astype(jnp.int32)

    dpad = N + jnp.arange(E_DPAD - 4 * E, dtype=jnp.int32) % (NP - N)
    deg_idx = jnp.concatenate(
        [src0, dst0 + NP, src1 + 2 * NP, dst1 + 3 * NP, dpad]
    ).reshape(NW, NBD, KD)
    degp = _deg_kernel(deg_idx)
    deg = degp.sum((0, 1))
    rs = lax.rsqrt(jnp.maximum(deg, 1.0)).reshape(4, NP)
    ns0 = rs[0, :, None]
    nd0 = rs[1, :, None]
    ns1 = rs[2, :, None]
    nd1 = rs[3, :, None]

    espread = jnp.arange(E_PAD - E, dtype=jnp.int32) % (NP - N)
    epad_s = PAD_SRC + espread
    epad_d = PAD_SRC + espread
    src0r = jnp.concatenate([src0, epad_s]).reshape(NCHUNK, CH, KB)
    dst0r = jnp.concatenate([dst0, epad_d]).reshape(NCHUNK, CH, KB)
    src1r = jnp.concatenate([src1, epad_s]).reshape(NCHUNK, CH, KB)
    dst1r = jnp.concatenate([dst1, epad_d]).reshape(NCHUNK, CH, KB)

    b1_r0_2d = b1_r0[None, :]
    b1_r1_2d = b1_r1[None, :]
    b2_r0_2d = b2_r0[None, :]
    b2_r1_2d = b2_r1[None, :]

    xp = jnp.pad(x, ((0, NP - N), (0, 0)))
    xns0, xns1 = _scale_call(xp, ns0, ns1)
    agg1 = _spmm_kernel(xns0, xns1, src0r, dst0r, src1r, dst1r)
    agg1 = agg1.reshape(2, NC, NP, D)
    hns0, hns1 = _layer1_call(agg1, nd0, nd1, ns0, ns1,
                              W1_r0, W1_r1, b1_r0_2d, b1_r1_2d)
    agg2 = _spmm_kernel(hns0, hns1, src0r, dst0r, src1r, dst1r)
    agg2 = agg2.reshape(2, NC, NP, D)
    outp = _layer2_call(agg2, nd0, nd1, W2_r0, W2_r1, b2_r0_2d, b2_r1_2d)
    return outp[:N]
